# baseline (speedup 1.0000x reference)
import functools

import jax
import jax.numpy as jnp
from jax import lax
from jax.experimental import pallas as pl
from jax.experimental.pallas import tpu as pltpu
from jax.experimental.pallas import tpu_sc as plsc

_K = 1024
_D = 64
_CCOST = 0.25


def _vq_idx_block(x_ref, cb_ref, idx_ref, loss_ref):
    x = x_ref[0]
    cb = cb_ref[...]
    csq = jnp.sum(cb * cb, axis=1, keepdims=True)
    xsq = jnp.sum(x * x, axis=0)
    prod = jax.lax.dot_general(
        cb, x, (((1,), (0,)), ((), ())),
        preferred_element_type=jnp.float32)
    score = (xsq[None, :] + csq) - 2.0 * prod
    m = jnp.min(score, axis=0)
    kiota = jax.lax.broadcasted_iota(jnp.int32, score.shape, 0)
    idx = jnp.min(jnp.where(score == m[None, :], kiota, _K), axis=0)
    idx_ref[0, 0] = idx
    loss_ref[0, 0] = jnp.broadcast_to(jnp.sum(m), (128,))


def _make_sc_gather(B, C, T):
    mesh = plsc.VectorSubcoreMesh(core_axis_name="c", subcore_axis_name="s")
    nparts = 4
    part = T // nparts

    @functools.partial(
        pl.kernel,
        out_type=jax.ShapeDtypeStruct((B, C, T), jnp.float32),
        mesh=mesh,
        scratch_types=[
            pltpu.VMEM((_K, _D), jnp.float32),
            pltpu.VMEM((T,), jnp.int32),
            pltpu.VMEM((C, part), jnp.float32),
        ],
        compiler_params=pltpu.CompilerParams(
            needs_layout_passes=False, use_tc_tiling_on_sc=False),
    )
    def sc_gather(cb_hbm, idx_hbm, out_hbm, cb_v, idx_v, buf_v):
        ncores = lax.axis_index("c")
        wid = lax.axis_index("s") * 2 + ncores

        @pl.when(wid < B)
        def _():
            pltpu.sync_copy(cb_hbm, cb_v)
            pltpu.sync_copy(idx_hbm.at[wid], idx_v)
            for h in range(nparts):
                def chunk(tc, carry):
                    t0 = h * part + tc * 16
                    rows = idx_v[pl.ds(t0, 16)]
                    for c in range(C):
                        col = jnp.full((16,), c, jnp.int32)
                        vals = plsc.load_gather(cb_v, [rows, col])
                        buf_v[c, pl.ds(tc * 16, 16)] = vals
                    return carry
                lax.fori_loop(0, part // 16, chunk, 0, unroll=False)
                pltpu.sync_copy(
                    buf_v, out_hbm.at[wid, :, pl.ds(h * part, part)])

    return sc_gather


def kernel(inputs, codebook):
    B, C, H, W = inputs.shape
    T = H * W
    x3 = inputs.reshape(B, C, T)
    idx, loss = pl.pallas_call(
        _vq_idx_block,
        grid=(B,),
        in_specs=[
            pl.BlockSpec((1, C, T), lambda b: (b, 0, 0)),
            pl.BlockSpec((_K, _D), lambda b: (0, 0)),
        ],
        out_specs=[
            pl.BlockSpec((1, 1, T), lambda b: (b, 0, 0)),
            pl.BlockSpec((1, 1, 128), lambda b: (b, 0, 0)),
        ],
        out_shape=[
            jax.ShapeDtypeStruct((B, 1, T), jnp.int32),
            jax.ShapeDtypeStruct((B, 1, 128), jnp.float32),
        ],
        compiler_params=pltpu.CompilerParams(
            dimension_semantics=("parallel",)),
    )(x3, codebook)
    idx2 = idx.reshape(B, T)
    quant = _make_sc_gather(B, C, T)(codebook, idx2)
    quantized_out = quant.reshape(B, C, H, W)
    encoding_indices = idx.reshape(B * T)
    vq_loss = (1.0 + _CCOST) * jnp.sum(loss[:, 0, 0]) / (B * C * T)
    return quantized_out, vq_loss, encoding_indices
